# baseline (speedup 1.0000x reference)
import functools

import jax
import jax.numpy as jnp
import numpy as np
from jax import lax
from jax.experimental import pallas as pl
from jax.experimental.pallas import tpu as pltpu
from jax.experimental.pallas import tpu_sc as plsc

_CUTOFF = 5.0
_B = 64
_GPAD = 64
_NPAD = 10240
_EPAD = 327680
_CHUNK = 64
_BLKE = 2048
_BLKN = 1024
_NW = 32
_EPW = _EPAD // _NW
_NCHUNK = _EPW // _CHUNK
_RPS = _NPAD // 16

def _sp(x):
    return jnp.maximum(x, 0.0) + jnp.log1p(jnp.exp(-jnp.abs(x)))


def _bdot(a, b):
    return jnp.dot(a.astype(jnp.bfloat16), b.astype(jnp.bfloat16),
                   preferred_element_type=jnp.float32)


def _filters_body(ew_ref, wf1_ref, bf1_ref, wf2_ref, bf2_ref, o_ref, *, E, G):
    j = pl.program_id(0)
    step = _CUTOFF / (G - 1)
    coef = -0.5 / step ** 2
    ew = ew_ref[...]
    off = lax.broadcasted_iota(jnp.int32, (1, _GPAD), 1).astype(jnp.float32)
    e = jnp.exp(coef * (ew - off * step) ** 2)
    w = _sp(_bdot(e, wf1_ref[...]) + bf1_ref[...])
    w = _sp(_bdot(w, wf2_ref[...]) + bf2_ref[...])
    row = j * _BLKE + lax.broadcasted_iota(jnp.int32, (_BLKE, 1), 0)
    o_ref[...] = jnp.where(row < E, w, 0.0)


def _filters(ew2, Wf1l, bf1l, Wf2l, bf2l, E, G, H):
    return pl.pallas_call(
        functools.partial(_filters_body, E=E, G=G),
        grid=(_EPAD // _BLKE,),
        in_specs=[
            pl.BlockSpec((_BLKE, 1), lambda j: (j, 0)),
            pl.BlockSpec((_GPAD, H), lambda j: (0, 0)),
            pl.BlockSpec((1, H), lambda j: (0, 0)),
            pl.BlockSpec((H, H), lambda j: (0, 0)),
            pl.BlockSpec((1, H), lambda j: (0, 0)),
        ],
        out_specs=pl.BlockSpec((_BLKE, H), lambda j: (j, 0)),
        out_shape=jax.ShapeDtypeStruct((_EPAD, H), jnp.float32),
    )(ew2, Wf1l, bf1l, Wf2l, bf2l)


def _embed_body(z_ref, emb_ref, wl1_ref, h_ref, x_ref):
    zb = z_ref[0, 0, :]
    rows = lax.broadcasted_iota(jnp.int32, (128, _BLKN), 0)
    oht = jnp.where(rows == zb[None, :], 1.0, 0.0)
    h = lax.dot_general(oht, emb_ref[...], (((0,), (0,)), ((), ())),
                        preferred_element_type=jnp.float32)
    h_ref[...] = h
    x_ref[...] = _bdot(h, wl1_ref[...])


def _embed(zp, embp, Wl1p0, H):
    nb = _NPAD // _BLKN
    return pl.pallas_call(
        _embed_body,
        grid=(nb,),
        in_specs=[
            pl.BlockSpec((1, 1, _BLKN), lambda j: (j, 0, 0)),
            pl.BlockSpec((128, H), lambda j: (0, 0)),
            pl.BlockSpec((H, H), lambda j: (0, 0)),
        ],
        out_specs=[
            pl.BlockSpec((_BLKN, H), lambda j: (j, 0)),
            pl.BlockSpec((_BLKN, H), lambda j: (j, 0)),
        ],
        out_shape=[
            jax.ShapeDtypeStruct((_NPAD, H), jnp.float32),
            jax.ShapeDtypeStruct((_NPAD, H), jnp.float32),
        ],
    )(zp, embp, Wl1p0)


def _sc_body(x_hbm, w_hbm, src_hbm, dst_hbm, out_hbm,
             xr0, xr1, xr2, xr3, wr,
             si0, si1, si2, si3, di0, di1, di2, di3, agg,
             gs0, gs1, ws, ss0, ss1, ss2, ss3, is0, is1, is2, is3):
    c = lax.axis_index("c")
    s = lax.axis_index("s")
    wid = c * 16 + s
    xr = (xr0, xr1, xr2, xr3)
    sidx = (si0, si1, si2, si3)
    didx = (di0, di1, di2, di3)
    gsem = (gs0, gs1)
    ssem = (ss0, ss1, ss2, ss3)
    isem = (is0, is1, is2, is3)

    def _zrow(i, carry):
        for jj in range(8):
            xr0[i, pl.ds(jj * 16, 16)] = jnp.zeros((16,), jnp.float32)
        return carry
    lax.fori_loop(0, _CHUNK, _zrow, 0)
    for k in range(0, _RPS, _CHUNK):
        pltpu.sync_copy(xr0, agg.at[pl.ds(s * _RPS + k, _CHUNK)])
    plsc.subcore_barrier()

    def _issue_idx(g, q):
        pltpu.async_copy(src_hbm.at[wid, g], sidx[q], isem[q])
        pltpu.async_copy(dst_hbm.at[wid, g], didx[q], isem[q])

    def _wait_idx(g, q):
        pltpu.make_async_copy(src_hbm.at[wid, g], sidx[q], isem[q]).wait()
        pltpu.make_async_copy(dst_hbm.at[wid, g], didx[q], isem[q]).wait()

    _issue_idx(0, 0)
    _issue_idx(1, 1)
    _wait_idx(0, 0)
    pltpu.async_copy(x_hbm.at[sidx[0]], xr[0], gsem[0])
    pltpu.async_copy(w_hbm.at[wid, 0], wr, ws)

    def _outer(g4, carry):
        g0 = g4 * 4
        for u in range(4):
            g = g0 + u
            b = u % 2
            o = b ^ 1

            @pl.when(g >= 2)
            def _():
                pltpu.make_async_copy(xr[(u + 2) % 4],
                                      agg.at[didx[(u + 2) % 4]],
                                      ssem[(u + 2) % 4]).wait()

            @pl.when(g + 2 < _NCHUNK)
            def _():
                _issue_idx(g + 2, (u + 2) % 4)

            @pl.when(g + 1 < _NCHUNK)
            def _():
                _wait_idx(g + 1, (u + 1) % 4)
                pltpu.async_copy(x_hbm.at[sidx[(u + 1) % 4]],
                                 xr[(u + 1) % 4], gsem[o])

            pltpu.make_async_copy(x_hbm.at[sidx[u]], xr[u], gsem[b]).wait()
            pltpu.make_async_copy(w_hbm.at[wid, g], wr, ws).wait()

            def _mrow(i, cc):
                for jj in range(8):
                    sl = pl.ds(jj * 16, 16)
                    xr[u][i, sl] = xr[u][i, sl] * wr[i, sl]
                return cc
            lax.fori_loop(0, _CHUNK, _mrow, 0)

            @pl.when(g + 1 < _NCHUNK)
            def _():
                pltpu.async_copy(w_hbm.at[wid, g + 1], wr, ws)
            pltpu.async_copy(xr[u], agg.at[didx[u]], ssem[u], add=True)
        return carry

    lax.fori_loop(0, _NCHUNK // 4, _outer, 0)
    for u in ((_NCHUNK - 2) % 4, (_NCHUNK - 1) % 4):
        pltpu.make_async_copy(xr[u], agg.at[didx[u]], ssem[u]).wait()
    plsc.subcore_barrier()

    for k in range(0, _RPS, _CHUNK):
        off = s * _RPS + k
        pltpu.sync_copy(agg.at[pl.ds(off, _CHUNK)],
                        out_hbm.at[c, pl.ds(off, _CHUNK)])


def _gather_mul_scatter(x, w, src3, dst3, H):
    mesh = plsc.VectorSubcoreMesh(core_axis_name="c", subcore_axis_name="s",
                                  num_cores=2, num_subcores=16)
    dma = pltpu.SemaphoreType.DMA
    return pl.kernel(
        _sc_body,
        out_type=jax.ShapeDtypeStruct((2, _NPAD, H), jnp.float32),
        mesh=mesh,
        scratch_types=(
            [pltpu.VMEM((_CHUNK, H), jnp.float32)] * 5
            + [pltpu.VMEM((_CHUNK,), jnp.int32)] * 8
            + [pltpu.VMEM_SHARED((_NPAD, H), jnp.float32)]
            + [dma] * 11
        ),
    )(x, w.reshape(_NW, _NCHUNK, _CHUNK, H), src3, dst3)


def _update_body(p_ref, h_ref, wl2_ref, bl2_ref, wl3_ref, bl3_ref, wln_ref,
                 hn_ref, xn_ref):
    agg = p_ref[0] + p_ref[1]
    t = _sp(jnp.dot(agg, wl2_ref[...], preferred_element_type=jnp.float32)
            + bl2_ref[...])
    t = jnp.dot(t, wl3_ref[...], preferred_element_type=jnp.float32) \
        + bl3_ref[...]
    hn = h_ref[...] + t
    hn_ref[...] = hn
    xn_ref[...] = _bdot(hn, wln_ref[...])


def _node_update(p, h, Wl2l, bl2l, Wl3l, bl3l, Wl1n, H):
    nb = _NPAD // _BLKN
    return pl.pallas_call(
        _update_body,
        grid=(nb,),
        in_specs=[
            pl.BlockSpec((2, _BLKN, H), lambda j: (0, j, 0)),
            pl.BlockSpec((_BLKN, H), lambda j: (j, 0)),
            pl.BlockSpec((H, H), lambda j: (0, 0)),
            pl.BlockSpec((1, H), lambda j: (0, 0)),
            pl.BlockSpec((H, H), lambda j: (0, 0)),
            pl.BlockSpec((1, H), lambda j: (0, 0)),
            pl.BlockSpec((H, H), lambda j: (0, 0)),
        ],
        out_specs=[
            pl.BlockSpec((_BLKN, H), lambda j: (j, 0)),
            pl.BlockSpec((_BLKN, H), lambda j: (j, 0)),
        ],
        out_shape=[
            jax.ShapeDtypeStruct((_NPAD, H), jnp.float32),
            jax.ShapeDtypeStruct((_NPAD, H), jnp.float32),
        ],
    )(p, h, Wl2l, bl2l, Wl3l, bl3l, Wl1n)


def _readout_body(b_ref, h_ref, wro1_ref, bro1_ref, wro2_ref, bro2_ref,
                  o_ref, pool_acc, cnt_acc):
    j = pl.program_id(0)

    @pl.when(j == 0)
    def _():
        pool_acc[...] = jnp.zeros_like(pool_acc)
        cnt_acc[...] = jnp.zeros_like(cnt_acc)

    bb = b_ref[0, 0, :]
    rows = lax.broadcasted_iota(jnp.int32, (_B, _BLKN), 0)
    oht = jnp.where(rows == bb[None, :], 1.0, 0.0)
    pool_acc[...] += jnp.dot(oht, h_ref[...],
                             preferred_element_type=jnp.float32)
    cnt_acc[...] += jnp.broadcast_to(
        jnp.sum(oht, axis=1, keepdims=True), cnt_acc.shape)

    @pl.when(j == pl.num_programs(0) - 1)
    def _():
        pooled = pool_acc[...] / jnp.maximum(cnt_acc[...], 1.0)
        y = _sp(jnp.dot(_sp(pooled), wro1_ref[...],
                        preferred_element_type=jnp.float32) + bro1_ref[...])
        o_ref[...] = jnp.dot(y, wro2_ref[...],
                             preferred_element_type=jnp.float32) + bro2_ref[...]


def _readout(bp, h, Wro1p, bro1p, Wro2p, bro2p, H):
    nb = _NPAD // _BLKN
    return pl.pallas_call(
        _readout_body,
        grid=(nb,),
        in_specs=[
            pl.BlockSpec((1, 1, _BLKN), lambda j: (j, 0, 0)),
            pl.BlockSpec((_BLKN, H), lambda j: (j, 0)),
            pl.BlockSpec((H, H), lambda j: (0, 0)),
            pl.BlockSpec((1, H), lambda j: (0, 0)),
            pl.BlockSpec((H, H), lambda j: (0, 0)),
            pl.BlockSpec((1, H), lambda j: (0, 0)),
        ],
        out_specs=pl.BlockSpec((_B, H), lambda j: (0, 0)),
        out_shape=jax.ShapeDtypeStruct((_B, H), jnp.float32),
        scratch_shapes=[
            pltpu.VMEM((_B, H), jnp.float32),
            pltpu.VMEM((_B, H), jnp.float32),
        ],
    )(bp, h, Wro1p, bro1p, Wro2p, bro2p)


def kernel(z, edge_index, edge_weight, batch, emb, Wf1, bf1, Wf2, bf2,
           Wl1, Wl2, bl2, Wl3, bl3, Wro1, bro1, Wro2, bro2):
    N = z.shape[0]
    E = edge_weight.shape[0]
    V, H = emb.shape
    L, G, _ = Wf1.shape

    ew2 = jnp.pad(edge_weight, (0, _EPAD - E))[:, None]
    srcp = jnp.pad(edge_index[0], (0, _EPAD - E)).astype(jnp.int32)
    dstp = jnp.pad(edge_index[1], (0, _EPAD - E),
                   constant_values=_NPAD - 1).astype(jnp.int32)
    src3 = srcp.reshape(_NW, _NCHUNK, _CHUNK)
    dst3 = dstp.reshape(_NW, _NCHUNK, _CHUNK)
    zp = jnp.pad(z, (0, _NPAD - N)).astype(jnp.int32)
    zp = zp.reshape(_NPAD // _BLKN, 1, _BLKN)
    bp = jnp.pad(batch, (0, _NPAD - N),
                 constant_values=_B).astype(jnp.int32)
    bp = bp.reshape(_NPAD // _BLKN, 1, _BLKN)
    embp = jnp.pad(emb, ((0, 128 - V), (0, 0)))
    Wf1p = jnp.pad(Wf1, ((0, 0), (0, _GPAD - G), (0, 0)))
    Wf2p = Wf2
    bf2p = bf2
    Wro1p = jnp.pad(Wro1, ((0, 0), (0, H - Wro1.shape[1])))
    bro1p = jnp.pad(bro1, (0, H - bro1.shape[0]))[None, :]
    Wro2p = jnp.pad(Wro2, ((0, H - Wro2.shape[0]), (0, H - Wro2.shape[1])))
    bro2p = jnp.broadcast_to(bro2[None, :], (1, H))

    W_lyr = [_filters(ew2, Wf1p[l], bf1[l][None, :], Wf2p[l],
                      bf2p[l][None, :], E, G, H) for l in range(L)]
    h, x = _embed(zp, embp, Wl1[0], H)
    for l in range(L):
        p = _gather_mul_scatter(x, W_lyr[l], src3, dst3, H)
        h, x = _node_update(p, h, Wl2[l], bl2[l][None, :], Wl3[l],
                            bl3[l][None, :], Wl1[(l + 1) % L], H)
    out = _readout(bp, h, Wro1p, bro1p, Wro2p, bro2p, H)
    return out[:, :1]


# speedup vs baseline: 1.0154x; 1.0154x over previous
import functools

import jax
import jax.numpy as jnp
import numpy as np
from jax import lax
from jax.experimental import pallas as pl
from jax.experimental.pallas import tpu as pltpu
from jax.experimental.pallas import tpu_sc as plsc

_CUTOFF = 5.0
_B = 64
_GPAD = 64
_NPAD = 10240
_EPAD = 327680
_CHUNK = 64
_NAGG = 10112
_BLKE = 2048
_BLKN = 1024
_NW = 32
_EPW = _EPAD // _NW
_NCHUNK = _EPW // _CHUNK
_RPS = _NAGG // 16

def _sp(x):
    return jnp.maximum(x, 0.0) + jnp.log1p(jnp.exp(-jnp.abs(x)))


def _bdot(a, b):
    return jnp.dot(a.astype(jnp.bfloat16), b.astype(jnp.bfloat16),
                   preferred_element_type=jnp.float32)


def _filters_body(ew_ref, wf1_ref, bf1_ref, wf2_ref, bf2_ref, o_ref, *, E, G):
    j = pl.program_id(0)
    step = _CUTOFF / (G - 1)
    coef = -0.5 / step ** 2
    ew = ew_ref[...]
    off = lax.broadcasted_iota(jnp.int32, (1, _GPAD), 1).astype(jnp.float32)
    e = jnp.exp(coef * (ew - off * step) ** 2)
    w = _sp(_bdot(e, wf1_ref[...]) + bf1_ref[...])
    w = _sp(_bdot(w, wf2_ref[...]) + bf2_ref[...])
    row = j * _BLKE + lax.broadcasted_iota(jnp.int32, (_BLKE, 1), 0)
    o_ref[...] = jnp.where(row < E, w, 0.0)


def _filters(ew2, Wf1l, bf1l, Wf2l, bf2l, E, G, H):
    return pl.pallas_call(
        functools.partial(_filters_body, E=E, G=G),
        grid=(_EPAD // _BLKE,),
        in_specs=[
            pl.BlockSpec((_BLKE, 1), lambda j: (j, 0)),
            pl.BlockSpec((_GPAD, H), lambda j: (0, 0)),
            pl.BlockSpec((1, H), lambda j: (0, 0)),
            pl.BlockSpec((H, H), lambda j: (0, 0)),
            pl.BlockSpec((1, H), lambda j: (0, 0)),
        ],
        out_specs=pl.BlockSpec((_BLKE, H), lambda j: (j, 0)),
        out_shape=jax.ShapeDtypeStruct((_EPAD, H), jnp.float32),
    )(ew2, Wf1l, bf1l, Wf2l, bf2l)


def _embed_body(z_ref, emb_ref, wl1_ref, h_ref, x_ref):
    zb = z_ref[0, 0, :]
    rows = lax.broadcasted_iota(jnp.int32, (128, _BLKN), 0)
    oht = jnp.where(rows == zb[None, :], 1.0, 0.0)
    h = lax.dot_general(oht, emb_ref[...], (((0,), (0,)), ((), ())),
                        preferred_element_type=jnp.float32)
    h_ref[...] = h
    x_ref[...] = _bdot(h, wl1_ref[...])


def _embed(zp, embp, Wl1p0, H):
    nb = _NPAD // _BLKN
    return pl.pallas_call(
        _embed_body,
        grid=(nb,),
        in_specs=[
            pl.BlockSpec((1, 1, _BLKN), lambda j: (j, 0, 0)),
            pl.BlockSpec((128, H), lambda j: (0, 0)),
            pl.BlockSpec((H, H), lambda j: (0, 0)),
        ],
        out_specs=[
            pl.BlockSpec((_BLKN, H), lambda j: (j, 0)),
            pl.BlockSpec((_BLKN, H), lambda j: (j, 0)),
        ],
        out_shape=[
            jax.ShapeDtypeStruct((_NPAD, H), jnp.float32),
            jax.ShapeDtypeStruct((_NPAD, H), jnp.float32),
        ],
    )(zp, embp, Wl1p0)


def _sc_body(x_hbm, w_hbm, src_hbm, dst_hbm, out_hbm,
             xr0, xr1, wr0, wr1, pr0, pr1,
             si0, si1, si2, si3, di0, di1, di2, di3, agg,
             gs0, gs1, ws0, ws1, ss0, ss1, is0, is1, is2, is3):
    c = lax.axis_index("c")
    s = lax.axis_index("s")
    wid = c * 16 + s
    xr = (xr0, xr1)
    wr = (wr0, wr1)
    pr = (pr0, pr1)
    sidx = (si0, si1, si2, si3)
    didx = (di0, di1, di2, di3)
    gsem = (gs0, gs1)
    wsem = (ws0, ws1)
    ssem = (ss0, ss1)
    isem = (is0, is1, is2, is3)

    def _zrow(i, carry):
        for jj in range(8):
            xr0[i, pl.ds(jj * 16, 16)] = jnp.zeros((16,), jnp.float32)
        return carry
    lax.fori_loop(0, _CHUNK, _zrow, 0)
    for k in range(0, _RPS - _CHUNK + 1, _CHUNK):
        pltpu.sync_copy(xr0, agg.at[pl.ds(s * _RPS + k, _CHUNK)])
    if _RPS % _CHUNK:
        r = _RPS % _CHUNK
        pltpu.sync_copy(xr0.at[pl.ds(0, r)],
                        agg.at[pl.ds(s * _RPS + _RPS - r, r)])
    plsc.subcore_barrier()

    def _issue_idx(g, q):
        pltpu.async_copy(src_hbm.at[wid, g], sidx[q], isem[q])
        pltpu.async_copy(dst_hbm.at[wid, g], didx[q], isem[q])

    def _wait_idx(g, q):
        pltpu.make_async_copy(src_hbm.at[wid, g], sidx[q], isem[q]).wait()
        pltpu.make_async_copy(dst_hbm.at[wid, g], didx[q], isem[q]).wait()

    def _issue_data(g, b, q):
        pltpu.async_copy(x_hbm.at[sidx[q]], xr[b], gsem[b])
        pltpu.async_copy(w_hbm.at[wid, g], wr[b], wsem[b])

    _issue_idx(0, 0)
    _issue_idx(1, 1)
    _wait_idx(0, 0)
    _issue_data(0, 0, 0)

    def _outer(g4, carry):
        g0 = g4 * 4
        for u in range(4):
            g = g0 + u
            b = u % 2
            o = b ^ 1
            q = u % 4

            @pl.when(g >= 2)
            def _():
                pltpu.make_async_copy(pr[b], agg.at[didx[(q + 2) % 4]],
                                      ssem[b]).wait()

            @pl.when(g + 2 < _NCHUNK)
            def _():
                _issue_idx(g + 2, (q + 2) % 4)

            @pl.when(g + 1 < _NCHUNK)
            def _():
                _wait_idx(g + 1, (q + 1) % 4)
                _issue_data(g + 1, o, (q + 1) % 4)

            pltpu.make_async_copy(x_hbm.at[sidx[q]], xr[b], gsem[b]).wait()
            pltpu.make_async_copy(w_hbm.at[wid, g], wr[b], wsem[b]).wait()

            def _mrow(i, cc):
                for jj in range(8):
                    sl = pl.ds(jj * 16, 16)
                    pr[b][i, sl] = xr[b][i, sl] * wr[b][i, sl]
                return cc
            lax.fori_loop(0, _CHUNK, _mrow, 0)
            pltpu.async_copy(pr[b], agg.at[didx[q]], ssem[b], add=True)
        return carry

    lax.fori_loop(0, _NCHUNK // 4, _outer, 0)
    for b in range(2):
        pltpu.make_async_copy(pr[b], agg.at[didx[b]], ssem[b]).wait()
    plsc.subcore_barrier()

    for k in range(0, _RPS - _CHUNK + 1, _CHUNK):
        off = s * _RPS + k
        pltpu.sync_copy(agg.at[pl.ds(off, _CHUNK)],
                        out_hbm.at[c, pl.ds(off, _CHUNK)])
    if _RPS % _CHUNK:
        r = _RPS % _CHUNK
        off = s * _RPS + _RPS - r
        pltpu.sync_copy(agg.at[pl.ds(off, r)],
                        out_hbm.at[c, pl.ds(off, r)])


def _gather_mul_scatter(x, w, src3, dst3, H):
    mesh = plsc.VectorSubcoreMesh(core_axis_name="c", subcore_axis_name="s",
                                  num_cores=2, num_subcores=16)
    dma = pltpu.SemaphoreType.DMA
    return pl.kernel(
        _sc_body,
        out_type=jax.ShapeDtypeStruct((2, _NAGG, H), jnp.float32),
        mesh=mesh,
        scratch_types=(
            [pltpu.VMEM((_CHUNK, H), jnp.float32)] * 6
            + [pltpu.VMEM((_CHUNK,), jnp.int32)] * 8
            + [pltpu.VMEM_SHARED((_NAGG, H), jnp.float32)]
            + [dma] * 10
        ),
    )(x, w.reshape(_NW, _NCHUNK, _CHUNK, H), src3, dst3)


def _update_body(p_ref, h_ref, wl2_ref, bl2_ref, wl3_ref, bl3_ref, wln_ref,
                 hn_ref, xn_ref):
    agg = p_ref[0] + p_ref[1]
    t = _sp(jnp.dot(agg, wl2_ref[...], preferred_element_type=jnp.float32)
            + bl2_ref[...])
    t = jnp.dot(t, wl3_ref[...], preferred_element_type=jnp.float32) \
        + bl3_ref[...]
    hn = h_ref[...] + t
    hn_ref[...] = hn
    xn_ref[...] = _bdot(hn, wln_ref[...])


def _node_update(p, h, Wl2l, bl2l, Wl3l, bl3l, Wl1n, H):
    nb = _NPAD // _BLKN
    return pl.pallas_call(
        _update_body,
        grid=(nb,),
        in_specs=[
            pl.BlockSpec((2, _BLKN, H), lambda j: (0, j, 0)),
            pl.BlockSpec((_BLKN, H), lambda j: (j, 0)),
            pl.BlockSpec((H, H), lambda j: (0, 0)),
            pl.BlockSpec((1, H), lambda j: (0, 0)),
            pl.BlockSpec((H, H), lambda j: (0, 0)),
            pl.BlockSpec((1, H), lambda j: (0, 0)),
            pl.BlockSpec((H, H), lambda j: (0, 0)),
        ],
        out_specs=[
            pl.BlockSpec((_BLKN, H), lambda j: (j, 0)),
            pl.BlockSpec((_BLKN, H), lambda j: (j, 0)),
        ],
        out_shape=[
            jax.ShapeDtypeStruct((_NPAD, H), jnp.float32),
            jax.ShapeDtypeStruct((_NPAD, H), jnp.float32),
        ],
    )(p, h, Wl2l, bl2l, Wl3l, bl3l, Wl1n)


def _readout_body(b_ref, h_ref, wro1_ref, bro1_ref, wro2_ref, bro2_ref,
                  o_ref, pool_acc, cnt_acc):
    j = pl.program_id(0)

    @pl.when(j == 0)
    def _():
        pool_acc[...] = jnp.zeros_like(pool_acc)
        cnt_acc[...] = jnp.zeros_like(cnt_acc)

    bb = b_ref[0, 0, :]
    rows = lax.broadcasted_iota(jnp.int32, (_B, _BLKN), 0)
    oht = jnp.where(rows == bb[None, :], 1.0, 0.0)
    pool_acc[...] += jnp.dot(oht, h_ref[...],
                             preferred_element_type=jnp.float32)
    cnt_acc[...] += jnp.broadcast_to(
        jnp.sum(oht, axis=1, keepdims=True), cnt_acc.shape)

    @pl.when(j == pl.num_programs(0) - 1)
    def _():
        pooled = pool_acc[...] / jnp.maximum(cnt_acc[...], 1.0)
        y = _sp(jnp.dot(_sp(pooled), wro1_ref[...],
                        preferred_element_type=jnp.float32) + bro1_ref[...])
        o_ref[...] = jnp.dot(y, wro2_ref[...],
                             preferred_element_type=jnp.float32) + bro2_ref[...]


def _readout(bp, h, Wro1p, bro1p, Wro2p, bro2p, H):
    nb = _NPAD // _BLKN
    return pl.pallas_call(
        _readout_body,
        grid=(nb,),
        in_specs=[
            pl.BlockSpec((1, 1, _BLKN), lambda j: (j, 0, 0)),
            pl.BlockSpec((_BLKN, H), lambda j: (j, 0)),
            pl.BlockSpec((H, H), lambda j: (0, 0)),
            pl.BlockSpec((1, H), lambda j: (0, 0)),
            pl.BlockSpec((H, H), lambda j: (0, 0)),
            pl.BlockSpec((1, H), lambda j: (0, 0)),
        ],
        out_specs=pl.BlockSpec((_B, H), lambda j: (0, 0)),
        out_shape=jax.ShapeDtypeStruct((_B, H), jnp.float32),
        scratch_shapes=[
            pltpu.VMEM((_B, H), jnp.float32),
            pltpu.VMEM((_B, H), jnp.float32),
        ],
    )(bp, h, Wro1p, bro1p, Wro2p, bro2p)


def kernel(z, edge_index, edge_weight, batch, emb, Wf1, bf1, Wf2, bf2,
           Wl1, Wl2, bl2, Wl3, bl3, Wro1, bro1, Wro2, bro2):
    N = z.shape[0]
    E = edge_weight.shape[0]
    V, H = emb.shape
    L, G, _ = Wf1.shape

    ew2 = jnp.pad(edge_weight, (0, _EPAD - E))[:, None]
    srcp = jnp.pad(edge_index[0], (0, _EPAD - E)).astype(jnp.int32)
    dstp = jnp.pad(edge_index[1], (0, _EPAD - E),
                   constant_values=_NAGG - 1).astype(jnp.int32)
    src3 = srcp.reshape(_NW, _NCHUNK, _CHUNK)
    dst3 = dstp.reshape(_NW, _NCHUNK, _CHUNK)
    zp = jnp.pad(z, (0, _NPAD - N)).astype(jnp.int32)
    zp = zp.reshape(_NPAD // _BLKN, 1, _BLKN)
    bp = jnp.pad(batch, (0, _NPAD - N),
                 constant_values=_B).astype(jnp.int32)
    bp = bp.reshape(_NPAD // _BLKN, 1, _BLKN)
    embp = jnp.pad(emb, ((0, 128 - V), (0, 0)))
    Wf1p = jnp.pad(Wf1, ((0, 0), (0, _GPAD - G), (0, 0)))
    Wf2p = Wf2
    bf2p = bf2
    Wro1p = jnp.pad(Wro1, ((0, 0), (0, H - Wro1.shape[1])))
    bro1p = jnp.pad(bro1, (0, H - bro1.shape[0]))[None, :]
    Wro2p = jnp.pad(Wro2, ((0, H - Wro2.shape[0]), (0, H - Wro2.shape[1])))
    bro2p = jnp.broadcast_to(bro2[None, :], (1, H))

    W_lyr = [_filters(ew2, Wf1p[l], bf1[l][None, :], Wf2p[l],
                      bf2p[l][None, :], E, G, H) for l in range(L)]
    h, x = _embed(zp, embp, Wl1[0], H)
    for l in range(L):
        p = _gather_mul_scatter(x, W_lyr[l], src3, dst3, H)
        p = jnp.pad(p, ((0, 0), (0, _NPAD - _NAGG), (0, 0)))
        h, x = _node_update(p, h, Wl2[l], bl2[l][None, :], Wl3[l],
                            bl3[l][None, :], Wl1[(l + 1) % L], H)
    out = _readout(bp, h, Wro1p, bro1p, Wro2p, bro2p, H)
    return out[:, :1]


# speedup vs baseline: 1.0881x; 1.0715x over previous
import functools

import jax
import jax.numpy as jnp
import numpy as np
from jax import lax
from jax.experimental import pallas as pl
from jax.experimental.pallas import tpu as pltpu
from jax.experimental.pallas import tpu_sc as plsc

_CUTOFF = 5.0
_B = 64
_GPAD = 64
_NPAD = 10240
_EPAD = 327680
_CHUNK = 40
_BLKE = 2048
_BLKN = 1024
_NW = 32
_EPW = _EPAD // _NW
_NCHUNK = _EPW // _CHUNK
_RPS = _NPAD // 16

def _sp(x):
    return jnp.maximum(x, 0.0) + jnp.log1p(jnp.exp(-jnp.abs(x)))


def _bdot(a, b):
    return jnp.dot(a.astype(jnp.bfloat16), b.astype(jnp.bfloat16),
                   preferred_element_type=jnp.float32)


def _filters_body(ew_ref, wf1_ref, bf1_ref, wf2_ref, bf2_ref, o_ref, *, E, G):
    j = pl.program_id(0)
    step = _CUTOFF / (G - 1)
    coef = -0.5 / step ** 2
    ew = ew_ref[...]
    off = lax.broadcasted_iota(jnp.int32, (1, _GPAD), 1).astype(jnp.float32)
    e = jnp.exp(coef * (ew - off * step) ** 2)
    w = _sp(_bdot(e, wf1_ref[...]) + bf1_ref[...])
    w = _sp(_bdot(w, wf2_ref[...]) + bf2_ref[...])
    row = j * _BLKE + lax.broadcasted_iota(jnp.int32, (_BLKE, 1), 0)
    o_ref[...] = jnp.where(row < E, w, 0.0)


def _filters(ew2, Wf1l, bf1l, Wf2l, bf2l, E, G, H):
    return pl.pallas_call(
        functools.partial(_filters_body, E=E, G=G),
        grid=(_EPAD // _BLKE,),
        in_specs=[
            pl.BlockSpec((_BLKE, 1), lambda j: (j, 0)),
            pl.BlockSpec((_GPAD, H), lambda j: (0, 0)),
            pl.BlockSpec((1, H), lambda j: (0, 0)),
            pl.BlockSpec((H, H), lambda j: (0, 0)),
            pl.BlockSpec((1, H), lambda j: (0, 0)),
        ],
        out_specs=pl.BlockSpec((_BLKE, H), lambda j: (j, 0)),
        out_shape=jax.ShapeDtypeStruct((_EPAD, H), jnp.float32),
    )(ew2, Wf1l, bf1l, Wf2l, bf2l)


def _embed_body(z_ref, emb_ref, wl1_ref, h_ref, x_ref):
    zb = z_ref[0, 0, :]
    rows = lax.broadcasted_iota(jnp.int32, (128, _BLKN), 0)
    oht = jnp.where(rows == zb[None, :], 1.0, 0.0)
    h = lax.dot_general(oht, emb_ref[...], (((0,), (0,)), ((), ())),
                        preferred_element_type=jnp.float32)
    h_ref[...] = h
    x_ref[...] = _bdot(h, wl1_ref[...])


def _embed(zp, embp, Wl1p0, H):
    nb = _NPAD // _BLKN
    return pl.pallas_call(
        _embed_body,
        grid=(nb,),
        in_specs=[
            pl.BlockSpec((1, 1, _BLKN), lambda j: (j, 0, 0)),
            pl.BlockSpec((128, H), lambda j: (0, 0)),
            pl.BlockSpec((H, H), lambda j: (0, 0)),
        ],
        out_specs=[
            pl.BlockSpec((_BLKN, H), lambda j: (j, 0)),
            pl.BlockSpec((_BLKN, H), lambda j: (j, 0)),
        ],
        out_shape=[
            jax.ShapeDtypeStruct((_NPAD, H), jnp.float32),
            jax.ShapeDtypeStruct((_NPAD, H), jnp.float32),
        ],
    )(zp, embp, Wl1p0)


def _sc_body(x_hbm, w_hbm, src_hbm, dst_hbm, out_hbm,
             xr0, xr1, wr0, wr1, pr0, pr1,
             si0, si1, si2, si3, di0, di1, di2, di3, agg,
             gs0, gs1, ws0, ws1, ss0, ss1, is0, is1, is2, is3):
    c = lax.axis_index("c")
    s = lax.axis_index("s")
    wid = c * 16 + s
    xr = (xr0, xr1)
    wr = (wr0, wr1)
    pr = (pr0, pr1)
    sidx = (si0, si1, si2, si3)
    didx = (di0, di1, di2, di3)
    gsem = (gs0, gs1)
    wsem = (ws0, ws1)
    ssem = (ss0, ss1)
    isem = (is0, is1, is2, is3)

    def _zrow(i, carry):
        for jj in range(8):
            xr0[i, pl.ds(jj * 16, 16)] = jnp.zeros((16,), jnp.float32)
        return carry
    lax.fori_loop(0, _CHUNK, _zrow, 0)
    for k in range(0, _RPS, _CHUNK):
        pltpu.sync_copy(xr0, agg.at[pl.ds(s * _RPS + k, _CHUNK)])
    plsc.subcore_barrier()

    def _issue_idx(g, q):
        pltpu.async_copy(src_hbm.at[wid, g], sidx[q], isem[q])
        pltpu.async_copy(dst_hbm.at[wid, g], didx[q], isem[q])

    def _wait_idx(g, q):
        pltpu.make_async_copy(src_hbm.at[wid, g], sidx[q], isem[q]).wait()
        pltpu.make_async_copy(dst_hbm.at[wid, g], didx[q], isem[q]).wait()

    def _issue_data(g, b, q):
        pltpu.async_copy(x_hbm.at[sidx[q]], xr[b], gsem[b])
        pltpu.async_copy(w_hbm.at[wid, g], wr[b], wsem[b])

    _issue_idx(0, 0)
    _issue_idx(1, 1)
    _wait_idx(0, 0)
    _issue_data(0, 0, 0)

    def _outer(g4, carry):
        g0 = g4 * 4
        for u in range(4):
            g = g0 + u
            b = u % 2
            o = b ^ 1
            q = u % 4

            @pl.when(g >= 2)
            def _():
                pltpu.make_async_copy(pr[b], agg.at[didx[(q + 2) % 4]],
                                      ssem[b]).wait()

            @pl.when(g + 2 < _NCHUNK)
            def _():
                _issue_idx(g + 2, (q + 2) % 4)

            @pl.when(g + 1 < _NCHUNK)
            def _():
                _wait_idx(g + 1, (q + 1) % 4)
                _issue_data(g + 1, o, (q + 1) % 4)

            pltpu.make_async_copy(x_hbm.at[sidx[q]], xr[b], gsem[b]).wait()
            pltpu.make_async_copy(w_hbm.at[wid, g], wr[b], wsem[b]).wait()

            def _mrow(i, cc):
                for jj in range(8):
                    sl = pl.ds(jj * 16, 16)
                    pr[b][i, sl] = xr[b][i, sl] * wr[b][i, sl]
                return cc
            lax.fori_loop(0, _CHUNK, _mrow, 0)
            pltpu.async_copy(pr[b], agg.at[didx[q]], ssem[b], add=True)
        return carry

    lax.fori_loop(0, _NCHUNK // 4, _outer, 0)
    for b in range(2):
        pltpu.make_async_copy(pr[b], agg.at[didx[b]], ssem[b]).wait()
    plsc.subcore_barrier()

    for k in range(0, _RPS, _CHUNK):
        off = s * _RPS + k
        pltpu.sync_copy(agg.at[pl.ds(off, _CHUNK)],
                        out_hbm.at[c, pl.ds(off, _CHUNK)])


def _gather_mul_scatter(x, w, src3, dst3, H):
    mesh = plsc.VectorSubcoreMesh(core_axis_name="c", subcore_axis_name="s",
                                  num_cores=2, num_subcores=16)
    dma = pltpu.SemaphoreType.DMA
    return pl.kernel(
        _sc_body,
        out_type=jax.ShapeDtypeStruct((2, _NPAD, H), jnp.float32),
        mesh=mesh,
        scratch_types=(
            [pltpu.VMEM((_CHUNK, H), jnp.float32)] * 6
            + [pltpu.VMEM((_CHUNK,), jnp.int32)] * 8
            + [pltpu.VMEM_SHARED((_NPAD, H), jnp.float32)]
            + [dma] * 10
        ),
    )(x, w.reshape(_NW, _NCHUNK, _CHUNK, H), src3, dst3)


def _update_body(p_ref, h_ref, wl2_ref, bl2_ref, wl3_ref, bl3_ref, wln_ref,
                 hn_ref, xn_ref):
    agg = p_ref[0] + p_ref[1]
    t = _sp(jnp.dot(agg, wl2_ref[...], preferred_element_type=jnp.float32)
            + bl2_ref[...])
    t = jnp.dot(t, wl3_ref[...], preferred_element_type=jnp.float32) \
        + bl3_ref[...]
    hn = h_ref[...] + t
    hn_ref[...] = hn
    xn_ref[...] = _bdot(hn, wln_ref[...])


def _node_update(p, h, Wl2l, bl2l, Wl3l, bl3l, Wl1n, H):
    nb = _NPAD // _BLKN
    return pl.pallas_call(
        _update_body,
        grid=(nb,),
        in_specs=[
            pl.BlockSpec((2, _BLKN, H), lambda j: (0, j, 0)),
            pl.BlockSpec((_BLKN, H), lambda j: (j, 0)),
            pl.BlockSpec((H, H), lambda j: (0, 0)),
            pl.BlockSpec((1, H), lambda j: (0, 0)),
            pl.BlockSpec((H, H), lambda j: (0, 0)),
            pl.BlockSpec((1, H), lambda j: (0, 0)),
            pl.BlockSpec((H, H), lambda j: (0, 0)),
        ],
        out_specs=[
            pl.BlockSpec((_BLKN, H), lambda j: (j, 0)),
            pl.BlockSpec((_BLKN, H), lambda j: (j, 0)),
        ],
        out_shape=[
            jax.ShapeDtypeStruct((_NPAD, H), jnp.float32),
            jax.ShapeDtypeStruct((_NPAD, H), jnp.float32),
        ],
    )(p, h, Wl2l, bl2l, Wl3l, bl3l, Wl1n)


def _readout_body(b_ref, h_ref, wro1_ref, bro1_ref, wro2_ref, bro2_ref,
                  o_ref, pool_acc, cnt_acc):
    j = pl.program_id(0)

    @pl.when(j == 0)
    def _():
        pool_acc[...] = jnp.zeros_like(pool_acc)
        cnt_acc[...] = jnp.zeros_like(cnt_acc)

    bb = b_ref[0, 0, :]
    rows = lax.broadcasted_iota(jnp.int32, (_B, _BLKN), 0)
    oht = jnp.where(rows == bb[None, :], 1.0, 0.0)
    pool_acc[...] += jnp.dot(oht, h_ref[...],
                             preferred_element_type=jnp.float32)
    cnt_acc[...] += jnp.broadcast_to(
        jnp.sum(oht, axis=1, keepdims=True), cnt_acc.shape)

    @pl.when(j == pl.num_programs(0) - 1)
    def _():
        pooled = pool_acc[...] / jnp.maximum(cnt_acc[...], 1.0)
        y = _sp(jnp.dot(_sp(pooled), wro1_ref[...],
                        preferred_element_type=jnp.float32) + bro1_ref[...])
        o_ref[...] = jnp.dot(y, wro2_ref[...],
                             preferred_element_type=jnp.float32) + bro2_ref[...]


def _readout(bp, h, Wro1p, bro1p, Wro2p, bro2p, H):
    nb = _NPAD // _BLKN
    return pl.pallas_call(
        _readout_body,
        grid=(nb,),
        in_specs=[
            pl.BlockSpec((1, 1, _BLKN), lambda j: (j, 0, 0)),
            pl.BlockSpec((_BLKN, H), lambda j: (j, 0)),
            pl.BlockSpec((H, H), lambda j: (0, 0)),
            pl.BlockSpec((1, H), lambda j: (0, 0)),
            pl.BlockSpec((H, H), lambda j: (0, 0)),
            pl.BlockSpec((1, H), lambda j: (0, 0)),
        ],
        out_specs=pl.BlockSpec((_B, H), lambda j: (0, 0)),
        out_shape=jax.ShapeDtypeStruct((_B, H), jnp.float32),
        scratch_shapes=[
            pltpu.VMEM((_B, H), jnp.float32),
            pltpu.VMEM((_B, H), jnp.float32),
        ],
    )(bp, h, Wro1p, bro1p, Wro2p, bro2p)


def kernel(z, edge_index, edge_weight, batch, emb, Wf1, bf1, Wf2, bf2,
           Wl1, Wl2, bl2, Wl3, bl3, Wro1, bro1, Wro2, bro2):
    N = z.shape[0]
    E = edge_weight.shape[0]
    V, H = emb.shape
    L, G, _ = Wf1.shape

    ew2 = jnp.pad(edge_weight, (0, _EPAD - E))[:, None]
    srcp = jnp.pad(edge_index[0], (0, _EPAD - E)).astype(jnp.int32)
    dstp = jnp.pad(edge_index[1], (0, _EPAD - E),
                   constant_values=_NPAD - 1).astype(jnp.int32)
    src3 = srcp.reshape(_NW, _NCHUNK, _CHUNK)
    dst3 = dstp.reshape(_NW, _NCHUNK, _CHUNK)
    zp = jnp.pad(z, (0, _NPAD - N)).astype(jnp.int32)
    zp = zp.reshape(_NPAD // _BLKN, 1, _BLKN)
    bp = jnp.pad(batch, (0, _NPAD - N),
                 constant_values=_B).astype(jnp.int32)
    bp = bp.reshape(_NPAD // _BLKN, 1, _BLKN)
    embp = jnp.pad(emb, ((0, 128 - V), (0, 0)))
    Wf1p = jnp.pad(Wf1, ((0, 0), (0, _GPAD - G), (0, 0)))
    Wf2p = Wf2
    bf2p = bf2
    Wro1p = jnp.pad(Wro1, ((0, 0), (0, H - Wro1.shape[1])))
    bro1p = jnp.pad(bro1, (0, H - bro1.shape[0]))[None, :]
    Wro2p = jnp.pad(Wro2, ((0, H - Wro2.shape[0]), (0, H - Wro2.shape[1])))
    bro2p = jnp.broadcast_to(bro2[None, :], (1, H))

    W_lyr = [_filters(ew2, Wf1p[l], bf1[l][None, :], Wf2p[l],
                      bf2p[l][None, :], E, G, H) for l in range(L)]
    h, x = _embed(zp, embp, Wl1[0], H)
    for l in range(L):
        p = _gather_mul_scatter(x, W_lyr[l], src3, dst3, H)
        h, x = _node_update(p, h, Wl2[l], bl2[l][None, :], Wl3[l],
                            bl3[l][None, :], Wl1[(l + 1) % L], H)
    out = _readout(bp, h, Wro1p, bro1p, Wro2p, bro2p, H)
    return out[:, :1]
